# Initial kernel scaffold; baseline (speedup 1.0000x reference)
import jax
import jax.numpy as jnp
from jax.experimental import pallas as pl


def kernel(x, emb_table, W, b):
    raise NotImplementedError("write your pallas kernel here")


# speedup vs baseline: 2.0500x; 2.0500x over previous
import jax
import jax.numpy as jnp
from jax import lax
from jax.experimental import pallas as pl
from jax.experimental.pallas import tpu as pltpu
from jax.experimental.pallas import tpu_sc as plsc

VOCAB = 51
EMB_DIM = 100
OUT_DIM = 51
BATCH = 4096
SEQ = 200

_PAD = 128
_N = BATCH * SEQ
_NW = 32
_PER_W = _N // _NW
_CHUNK = 128
_NB = _PER_W // _CHUNK


def _table_body(e_ref, w_ref, b_ref, out_ref):
    out_ref[...] = (
        jnp.dot(e_ref[...], w_ref[...], preferred_element_type=jnp.float32)
        + b_ref[...]
    )


def _fused_table(emb_table, W, b):
    w_pad = jnp.zeros((EMB_DIM, _PAD), jnp.float32).at[:, :OUT_DIM].set(W)
    b_pad = jnp.zeros((1, _PAD), jnp.float32).at[0, :OUT_DIM].set(b)
    return pl.pallas_call(
        _table_body,
        out_shape=jax.ShapeDtypeStruct((VOCAB, _PAD), jnp.float32),
    )(emb_table, w_pad, b_pad)


def _sc_body(table_hbm, idx_hbm, out_hbm, idx_v, rows_v, sem):
    wid = lax.axis_index("s") * 2 + lax.axis_index("c")
    base = wid * _PER_W
    pltpu.sync_copy(idx_hbm.at[pl.ds(base, _PER_W)], idx_v)

    def body(g, carry):
        start = g * _CHUNK
        pltpu.async_copy(
            table_hbm.at[idx_v.at[pl.ds(start, _CHUNK)]], rows_v, sem
        ).wait()
        pltpu.sync_copy(rows_v, out_hbm.at[pl.ds(base + start, _CHUNK)])
        return carry

    lax.fori_loop(0, _NB, body, 0)


def _sc_gather(table, xf):
    mesh = plsc.VectorSubcoreMesh(core_axis_name="c", subcore_axis_name="s")
    f = pl.kernel(
        _sc_body,
        mesh=mesh,
        out_type=jax.ShapeDtypeStruct((_N, _PAD), jnp.float32),
        scratch_types=[
            pltpu.VMEM((_PER_W,), jnp.int32),
            pltpu.VMEM((_CHUNK, _PAD), jnp.float32),
            pltpu.SemaphoreType.DMA,
        ],
    )
    return f(table, xf)


def kernel(x, emb_table, W, b):
    table = _fused_table(emb_table, W, b)
    xf = x.reshape(_N).astype(jnp.int32)
    out = _sc_gather(table, xf)
    return out[:, :OUT_DIM].reshape(BATCH, SEQ, OUT_DIM)


# speedup vs baseline: 4.1217x; 2.0106x over previous
import jax
import jax.numpy as jnp
from jax import lax
from jax.experimental import pallas as pl
from jax.experimental.pallas import tpu as pltpu
from jax.experimental.pallas import tpu_sc as plsc

VOCAB = 51
EMB_DIM = 100
OUT_DIM = 51
BATCH = 4096
SEQ = 200

_PAD = 128
_N = BATCH * SEQ
_NP = _N // 2
_V2 = VOCAB * VOCAB
_NW = 32
_PER_W = _NP // _NW
_CHUNK = 128
_NB = _PER_W // _CHUNK
_IDX_R = _NP // 128


def _prep_body(e_ref, w_ref, b_ref, xe_ref, xo_ref, t2_ref, idx_ref):
    t = (
        jnp.dot(e_ref[...], w_ref[...], preferred_element_type=jnp.float32)
        + b_ref[...]
    )
    rows = lax.broadcasted_iota(jnp.int32, (_V2, VOCAB), 0)
    cols = lax.broadcasted_iota(jnp.int32, (_V2, VOCAB), 1)
    one_a = (rows // VOCAB == cols).astype(jnp.float32)
    one_b = (rows % VOCAB == cols).astype(jnp.float32)
    left = jnp.dot(one_a, t, preferred_element_type=jnp.float32)
    right = jnp.dot(one_b, t, preferred_element_type=jnp.float32)
    pad = jnp.zeros((_V2, _PAD - 2 * OUT_DIM), jnp.float32)
    t2_ref[...] = jnp.concatenate([left, right, pad], axis=1)
    idx_ref[...] = xe_ref[...] * VOCAB + xo_ref[...]


def _prep(emb_table, W, b, xe, xo):
    return pl.pallas_call(
        _prep_body,
        out_shape=[
            jax.ShapeDtypeStruct((_V2, _PAD), jnp.float32),
            jax.ShapeDtypeStruct((_IDX_R, 128), jnp.int32),
        ],
    )(emb_table, W, b.reshape(1, OUT_DIM), xe, xo)


def _sc_body(table_hbm, idx_hbm, out_hbm, idx_v, rows_v, sem):
    wid = lax.axis_index("s") * 2 + lax.axis_index("c")
    base = wid * _PER_W
    pltpu.sync_copy(idx_hbm.at[pl.ds(base, _PER_W)], idx_v)

    def body(g, carry):
        start = g * _CHUNK
        pltpu.async_copy(
            table_hbm.at[idx_v.at[pl.ds(start, _CHUNK)]], rows_v, sem
        ).wait()
        pltpu.sync_copy(rows_v, out_hbm.at[pl.ds(base + start, _CHUNK)])
        return carry

    lax.fori_loop(0, _NB, body, 0)


def _sc_gather(table2, idx2):
    mesh = plsc.VectorSubcoreMesh(core_axis_name="c", subcore_axis_name="s")
    f = pl.kernel(
        _sc_body,
        mesh=mesh,
        out_type=jax.ShapeDtypeStruct((_NP, _PAD), jnp.float32),
        scratch_types=[
            pltpu.VMEM((_PER_W,), jnp.int32),
            pltpu.VMEM((_CHUNK, _PAD), jnp.float32),
            pltpu.SemaphoreType.DMA,
        ],
    )
    return f(table2, idx2)


def kernel(x, emb_table, W, b):
    xf = x.reshape(_NP, 2).astype(jnp.int32)
    xe = xf[:, 0].reshape(_IDX_R, 128)
    xo = xf[:, 1].reshape(_IDX_R, 128)
    table2, idx2 = _prep(emb_table, W, b, xe, xo)
    out = _sc_gather(table2, idx2.reshape(_NP))
    return out[:, : 2 * OUT_DIM].reshape(BATCH, SEQ, OUT_DIM)
